# Initial kernel scaffold; baseline (speedup 1.0000x reference)
import jax
import jax.numpy as jnp
from jax.experimental import pallas as pl


def kernel(word_embeddings, pe_table, ln_gamma, ln_beta):
    raise NotImplementedError("write your pallas kernel here")


# speedup vs baseline: 2.1475x; 2.1475x over previous
import jax
import jax.numpy as jnp
from jax.experimental import pallas as pl
from jax.experimental.pallas import tpu as pltpu

EPS_LN = 1e-12
BLOCK_S = 512


def _ln_body(we_ref, pe_ref, gamma_ref, beta_ref, out_ref):
    h = we_ref[0] + pe_ref[...]
    mean = jnp.mean(h, axis=-1, keepdims=True)
    c = h - mean
    var = jnp.mean(c * c, axis=-1, keepdims=True)
    inv = jax.lax.rsqrt(var + EPS_LN)
    out_ref[0] = c * inv * gamma_ref[...] + beta_ref[...]


def kernel(word_embeddings, pe_table, ln_gamma, ln_beta):
    B, S, D = word_embeddings.shape
    n_s = S // BLOCK_S
    gamma2 = ln_gamma.reshape(1, D)
    beta2 = ln_beta.reshape(1, D)
    return pl.pallas_call(
        _ln_body,
        grid=(n_s, B),
        in_specs=[
            pl.BlockSpec((1, BLOCK_S, D), lambda s, b: (b, s, 0)),
            pl.BlockSpec((BLOCK_S, D), lambda s, b: (s, 0)),
            pl.BlockSpec((1, D), lambda s, b: (0, 0)),
            pl.BlockSpec((1, D), lambda s, b: (0, 0)),
        ],
        out_specs=pl.BlockSpec((1, BLOCK_S, D), lambda s, b: (b, s, 0)),
        out_shape=jax.ShapeDtypeStruct((B, S, D), jnp.float32),
        compiler_params=pltpu.CompilerParams(
            dimension_semantics=("parallel", "parallel"),
        ),
    )(word_embeddings, pe_table, gamma2, beta2)


# speedup vs baseline: 2.4458x; 1.1389x over previous
import jax
import jax.numpy as jnp
from jax.experimental import pallas as pl
from jax.experimental.pallas import tpu as pltpu

EPS_LN = 1e-12
BLOCK_S = 1024


def _ln_body(we_ref, pe_ref, gamma_ref, beta_ref, out_ref):
    h = we_ref[0] + pe_ref[...]
    mean = jnp.mean(h, axis=-1, keepdims=True)
    c = h - mean
    var = jnp.mean(c * c, axis=-1, keepdims=True)
    inv = jax.lax.rsqrt(var + EPS_LN)
    out_ref[0] = c * inv * gamma_ref[...] + beta_ref[...]


def kernel(word_embeddings, pe_table, ln_gamma, ln_beta):
    B, S, D = word_embeddings.shape
    n_s = S // BLOCK_S
    gamma2 = ln_gamma.reshape(1, D)
    beta2 = ln_beta.reshape(1, D)
    return pl.pallas_call(
        _ln_body,
        grid=(n_s, B),
        in_specs=[
            pl.BlockSpec((1, BLOCK_S, D), lambda s, b: (b, s, 0)),
            pl.BlockSpec((BLOCK_S, D), lambda s, b: (s, 0)),
            pl.BlockSpec((1, D), lambda s, b: (0, 0)),
            pl.BlockSpec((1, D), lambda s, b: (0, 0)),
        ],
        out_specs=pl.BlockSpec((1, BLOCK_S, D), lambda s, b: (b, s, 0)),
        out_shape=jax.ShapeDtypeStruct((B, S, D), jnp.float32),
        compiler_params=pltpu.CompilerParams(
            dimension_semantics=("parallel", "parallel"),
        ),
    )(word_embeddings, pe_table, gamma2, beta2)


# speedup vs baseline: 2.5763x; 1.0534x over previous
import jax
import jax.numpy as jnp
from jax.experimental import pallas as pl
from jax.experimental.pallas import tpu as pltpu

EPS_LN = 1e-12
BLOCK_S = 2048


def _ln_body(we_ref, pe_ref, gamma_ref, beta_ref, out_ref):
    h = we_ref[0] + pe_ref[...]
    mean = jnp.mean(h, axis=-1, keepdims=True)
    c = h - mean
    var = jnp.mean(c * c, axis=-1, keepdims=True)
    inv = jax.lax.rsqrt(var + EPS_LN)
    out_ref[0] = c * inv * gamma_ref[...] + beta_ref[...]


def kernel(word_embeddings, pe_table, ln_gamma, ln_beta):
    B, S, D = word_embeddings.shape
    n_s = S // BLOCK_S
    gamma2 = ln_gamma.reshape(1, D)
    beta2 = ln_beta.reshape(1, D)
    return pl.pallas_call(
        _ln_body,
        grid=(n_s, B),
        in_specs=[
            pl.BlockSpec((1, BLOCK_S, D), lambda s, b: (b, s, 0)),
            pl.BlockSpec((BLOCK_S, D), lambda s, b: (s, 0)),
            pl.BlockSpec((1, D), lambda s, b: (0, 0)),
            pl.BlockSpec((1, D), lambda s, b: (0, 0)),
        ],
        out_specs=pl.BlockSpec((1, BLOCK_S, D), lambda s, b: (b, s, 0)),
        out_shape=jax.ShapeDtypeStruct((B, S, D), jnp.float32),
        compiler_params=pltpu.CompilerParams(
            dimension_semantics=("parallel", "parallel"),
        ),
    )(word_embeddings, pe_table, gamma2, beta2)
